# baseline (speedup 1.0000x reference)
import functools
import math

import jax
import jax.numpy as jnp
from jax.experimental import pallas as pl
from jax.experimental.pallas import tpu as pltpu

S = 2048
HIDDEN = 4096
E = 64
CAP = 64
BS = 256
NB = S // BS
EPS = float(jnp.finfo(jnp.float32).eps)


def _gate_stage(x_ref, w_ref, tri_ref,
                e1_ref, e2_ref, g1_ref, g2_ref, loc1_ref, loc2p_ref,
                cnt_ref, gsum_ref,
                carry1, carry2, gacc):
    b = pl.program_id(0)

    @pl.when(b == 0)
    def _init():
        carry1[...] = jnp.zeros_like(carry1)
        carry2[...] = jnp.zeros_like(carry2)
        gacc[...] = jnp.zeros_like(gacc)

    x = x_ref[...]
    w = w_ref[...]
    logits = jax.lax.dot_general(x, w, (((1,), (1,)), ((), ())),
                                 preferred_element_type=jnp.float32)
    m = jnp.max(logits, axis=1, keepdims=True)
    ex = jnp.exp(logits - m)
    gates = ex / jnp.sum(ex, axis=1, keepdims=True)

    eio = jax.lax.broadcasted_iota(jnp.int32, (BS, E), 1)
    g1 = jnp.max(gates, axis=1, keepdims=True)
    e1 = jnp.min(jnp.where(gates == g1, eio, E), axis=1, keepdims=True)
    hit1 = eio == e1
    gates_m = jnp.where(hit1, -1.0, gates)
    g2 = jnp.max(gates_m, axis=1, keepdims=True)
    e2 = jnp.min(jnp.where(gates_m == g2, eio, E), axis=1, keepdims=True)
    hit2 = eio == e2

    mask1 = hit1.astype(jnp.float32)
    mask2 = hit2.astype(jnp.float32)
    tri = tri_ref[...]
    cs1 = jax.lax.dot_general(tri, mask1, (((1,), (0,)), ((), ())),
                              preferred_element_type=jnp.float32)
    cs2 = jax.lax.dot_general(tri, mask2, (((1,), (0,)), ((), ())),
                              preferred_element_type=jnp.float32)
    c1 = carry1[0:1, :]
    c2 = carry2[0:1, :]
    loc1 = cs1 - 1.0 + c1
    loc2p = cs2 - 1.0 + c2

    e1_ref[...] = e1
    e2_ref[...] = e2
    g1_ref[...] = g1
    g2_ref[...] = g2
    loc1_ref[...] = jnp.sum(loc1 * mask1, axis=1, keepdims=True).astype(jnp.int32)
    loc2p_ref[...] = jnp.sum(loc2p * mask2, axis=1, keepdims=True).astype(jnp.int32)

    carry1[0:1, :] = c1 + cs1[BS - 1:BS, :]
    carry2[0:1, :] = c2 + cs2[BS - 1:BS, :]
    gacc[0:1, :] = gacc[0:1, :] + jnp.sum(gates, axis=0, keepdims=True)

    cnt_ref[...] = carry1[0:1, :].astype(jnp.int32)
    gsum_ref[...] = gacc[0:1, :]


def _dense_stage(e1_ref, e2_ref, g1_ref, g2_ref, loc1_ref, loc2p_ref,
                 cnt_ref, comb_ref, mask_ref):
    e1 = e1_ref[...]
    e2 = e2_ref[...]
    g1 = g1_ref[...]
    g2 = g2_ref[...]
    loc1 = loc1_ref[...]
    loc2p = loc2p_ref[...]
    cnt = cnt_ref[...]

    eio = jax.lax.broadcasted_iota(jnp.int32, (BS, E), 1)
    cnt2 = jnp.sum(jnp.where(eio == e2, cnt, 0), axis=1, keepdims=True)
    loc2 = loc2p + cnt2

    kept1 = loc1 < CAP
    kept2 = loc2 < CAP
    g1k = jnp.where(kept1, g1, 0.0)
    g2k = jnp.where(kept2, g2, 0.0)
    denom = jnp.maximum(g1k + g2k, EPS)
    c1 = g1k / denom
    c2 = g2k / denom

    cio = jax.lax.broadcasted_iota(jnp.int32, (BS, CAP), 1)
    a1 = jnp.where(eio == e1, c1, 0.0)
    a2 = jnp.where(eio == e2, c2, 0.0)
    b1 = (cio == loc1).astype(jnp.float32)
    b2 = (cio == loc2).astype(jnp.float32)
    comb = a1[:, :, None] * b1[:, None, :] + a2[:, :, None] * b2[:, None, :]
    comb_ref[...] = comb
    mask_ref[...] = comb != 0.0


@jax.jit
def kernel(input, W):
    x = input.astype(jnp.float32)
    tri = jnp.tril(jnp.ones((BS, BS), jnp.float32))

    outs = pl.pallas_call(
        _gate_stage,
        grid=(NB,),
        in_specs=[
            pl.BlockSpec((BS, HIDDEN), lambda b: (b, 0)),
            pl.BlockSpec((E, HIDDEN), lambda b: (0, 0)),
            pl.BlockSpec((BS, BS), lambda b: (0, 0)),
        ],
        out_specs=[
            pl.BlockSpec((BS, 1), lambda b: (b, 0)),
            pl.BlockSpec((BS, 1), lambda b: (b, 0)),
            pl.BlockSpec((BS, 1), lambda b: (b, 0)),
            pl.BlockSpec((BS, 1), lambda b: (b, 0)),
            pl.BlockSpec((BS, 1), lambda b: (b, 0)),
            pl.BlockSpec((BS, 1), lambda b: (b, 0)),
            pl.BlockSpec((1, E), lambda b: (0, 0)),
            pl.BlockSpec((1, E), lambda b: (0, 0)),
        ],
        out_shape=[
            jax.ShapeDtypeStruct((S, 1), jnp.int32),
            jax.ShapeDtypeStruct((S, 1), jnp.int32),
            jax.ShapeDtypeStruct((S, 1), jnp.float32),
            jax.ShapeDtypeStruct((S, 1), jnp.float32),
            jax.ShapeDtypeStruct((S, 1), jnp.int32),
            jax.ShapeDtypeStruct((S, 1), jnp.int32),
            jax.ShapeDtypeStruct((1, E), jnp.int32),
            jax.ShapeDtypeStruct((1, E), jnp.float32),
        ],
        scratch_shapes=[
            pltpu.VMEM((8, E), jnp.float32),
            pltpu.VMEM((8, E), jnp.float32),
            pltpu.VMEM((8, E), jnp.float32),
        ],
        compiler_params=pltpu.CompilerParams(
            dimension_semantics=("arbitrary",)),
    )(x, W, tri)
    e1, e2, g1, g2, loc1, loc2p, cnt, gsum = outs

    comb, mask = pl.pallas_call(
        _dense_stage,
        grid=(NB,),
        in_specs=[
            pl.BlockSpec((BS, 1), lambda b: (b, 0)),
            pl.BlockSpec((BS, 1), lambda b: (b, 0)),
            pl.BlockSpec((BS, 1), lambda b: (b, 0)),
            pl.BlockSpec((BS, 1), lambda b: (b, 0)),
            pl.BlockSpec((BS, 1), lambda b: (b, 0)),
            pl.BlockSpec((BS, 1), lambda b: (b, 0)),
            pl.BlockSpec((1, E), lambda b: (0, 0)),
        ],
        out_specs=[
            pl.BlockSpec((BS, E, CAP), lambda b: (b, 0, 0)),
            pl.BlockSpec((BS, E, CAP), lambda b: (b, 0, 0)),
        ],
        out_shape=[
            jax.ShapeDtypeStruct((S, E, CAP), jnp.float32),
            jax.ShapeDtypeStruct((S, E, CAP), jnp.bool_),
        ],
        compiler_params=pltpu.CompilerParams(
            dimension_semantics=("arbitrary",)),
    )(e1, e2, g1, g2, loc1, loc2p, cnt)

    exp_counts = cnt.reshape(E)
    me = gsum.reshape(E) / S
    ce = exp_counts.astype(jnp.float32) / S
    l_aux = jnp.mean(me * ce) * E * E
    return (l_aux, comb, mask, exp_counts)


# speedup vs baseline: 2.5038x; 2.5038x over previous
import functools
import math

import jax
import jax.numpy as jnp
from jax.experimental import pallas as pl
from jax.experimental.pallas import tpu as pltpu

S = 2048
HIDDEN = 4096
E = 64
CAP = 64
BS = 256
NB = S // BS
EPS = float(jnp.finfo(jnp.float32).eps)


def _gate_stage(x_ref, w_ref, tri_ref,
                e1_ref, e2_ref, g1_ref, g2_ref, loc1_ref, loc2p_ref,
                cnt_ref, gsum_ref,
                carry1, carry2, gacc):
    b = pl.program_id(0)

    @pl.when(b == 0)
    def _init():
        carry1[...] = jnp.zeros_like(carry1)
        carry2[...] = jnp.zeros_like(carry2)
        gacc[...] = jnp.zeros_like(gacc)

    x = x_ref[...]
    w = w_ref[...]
    logits = jax.lax.dot_general(x, w, (((1,), (1,)), ((), ())),
                                 preferred_element_type=jnp.float32)
    m = jnp.max(logits, axis=1, keepdims=True)
    ex = jnp.exp(logits - m)
    gates = ex / jnp.sum(ex, axis=1, keepdims=True)

    eio = jax.lax.broadcasted_iota(jnp.int32, (BS, E), 1)
    g1 = jnp.max(gates, axis=1, keepdims=True)
    e1 = jnp.min(jnp.where(gates == g1, eio, E), axis=1, keepdims=True)
    hit1 = eio == e1
    gates_m = jnp.where(hit1, -1.0, gates)
    g2 = jnp.max(gates_m, axis=1, keepdims=True)
    e2 = jnp.min(jnp.where(gates_m == g2, eio, E), axis=1, keepdims=True)
    hit2 = eio == e2

    mask1 = hit1.astype(jnp.float32)
    mask2 = hit2.astype(jnp.float32)
    tri = tri_ref[...]
    cs1 = jax.lax.dot_general(tri, mask1, (((1,), (0,)), ((), ())),
                              preferred_element_type=jnp.float32)
    cs2 = jax.lax.dot_general(tri, mask2, (((1,), (0,)), ((), ())),
                              preferred_element_type=jnp.float32)
    c1 = carry1[0:1, :]
    c2 = carry2[0:1, :]
    loc1 = cs1 - 1.0 + c1
    loc2p = cs2 - 1.0 + c2

    e1_ref[...] = e1
    e2_ref[...] = e2
    g1_ref[...] = g1
    g2_ref[...] = g2
    loc1_ref[...] = jnp.sum(loc1 * mask1, axis=1, keepdims=True).astype(jnp.int32)
    loc2p_ref[...] = jnp.sum(loc2p * mask2, axis=1, keepdims=True).astype(jnp.int32)

    carry1[0:1, :] = c1 + cs1[BS - 1:BS, :]
    carry2[0:1, :] = c2 + cs2[BS - 1:BS, :]
    gacc[0:1, :] = gacc[0:1, :] + jnp.sum(gates, axis=0, keepdims=True)

    cnt_ref[...] = carry1[0:1, :].astype(jnp.int32)
    gsum_ref[...] = gacc[0:1, :]


def _dense_stage(e1_ref, e2_ref, g1_ref, g2_ref, loc1_ref, loc2p_ref,
                 cnt_ref, comb_ref, mask_ref):
    e1 = e1_ref[...]
    e2 = e2_ref[...]
    g1 = g1_ref[...]
    g2 = g2_ref[...]
    loc1 = loc1_ref[...]
    loc2p = loc2p_ref[...]
    cnt = cnt_ref[...]

    eio = jax.lax.broadcasted_iota(jnp.int32, (E, BS), 0)
    cnt2 = jnp.sum(jnp.where(eio == e2, cnt, 0), axis=0, keepdims=True)
    loc2 = loc2p + cnt2

    kept1 = loc1 < CAP
    kept2 = loc2 < CAP
    g1k = jnp.where(kept1, g1, 0.0)
    g2k = jnp.where(kept2, g2, 0.0)
    denom = jnp.maximum(g1k + g2k, EPS)
    c1 = g1k / denom
    c2 = g2k / denom

    cio = jax.lax.broadcasted_iota(jnp.int32, (CAP, BS), 0)
    a1 = jnp.where(eio == e1, c1, 0.0)
    a2 = jnp.where(eio == e2, c2, 0.0)
    b1 = (cio == loc1).astype(jnp.float32)
    b2 = (cio == loc2).astype(jnp.float32)
    comb = a1[:, None, :] * b1[None, :, :] + a2[:, None, :] * b2[None, :, :]
    comb_ref[...] = comb
    mask_ref[...] = (comb != 0.0).astype(jnp.int8)


@jax.jit
def kernel(input, W):
    x = input.astype(jnp.float32)
    tri = jnp.tril(jnp.ones((BS, BS), jnp.float32))

    outs = pl.pallas_call(
        _gate_stage,
        grid=(NB,),
        in_specs=[
            pl.BlockSpec((BS, HIDDEN), lambda b: (b, 0)),
            pl.BlockSpec((E, HIDDEN), lambda b: (0, 0)),
            pl.BlockSpec((BS, BS), lambda b: (0, 0)),
        ],
        out_specs=[
            pl.BlockSpec((BS, 1), lambda b: (b, 0)),
            pl.BlockSpec((BS, 1), lambda b: (b, 0)),
            pl.BlockSpec((BS, 1), lambda b: (b, 0)),
            pl.BlockSpec((BS, 1), lambda b: (b, 0)),
            pl.BlockSpec((BS, 1), lambda b: (b, 0)),
            pl.BlockSpec((BS, 1), lambda b: (b, 0)),
            pl.BlockSpec((1, E), lambda b: (0, 0)),
            pl.BlockSpec((1, E), lambda b: (0, 0)),
        ],
        out_shape=[
            jax.ShapeDtypeStruct((S, 1), jnp.int32),
            jax.ShapeDtypeStruct((S, 1), jnp.int32),
            jax.ShapeDtypeStruct((S, 1), jnp.float32),
            jax.ShapeDtypeStruct((S, 1), jnp.float32),
            jax.ShapeDtypeStruct((S, 1), jnp.int32),
            jax.ShapeDtypeStruct((S, 1), jnp.int32),
            jax.ShapeDtypeStruct((1, E), jnp.int32),
            jax.ShapeDtypeStruct((1, E), jnp.float32),
        ],
        scratch_shapes=[
            pltpu.VMEM((8, E), jnp.float32),
            pltpu.VMEM((8, E), jnp.float32),
            pltpu.VMEM((8, E), jnp.float32),
        ],
        compiler_params=pltpu.CompilerParams(
            dimension_semantics=("arbitrary",)),
    )(x, W, tri)
    e1, e2, g1, g2, loc1, loc2p, cnt, gsum = outs

    e1t = e1.reshape(1, S)
    e2t = e2.reshape(1, S)
    g1t = g1.reshape(1, S)
    g2t = g2.reshape(1, S)
    loc1t = loc1.reshape(1, S)
    loc2pt = loc2p.reshape(1, S)
    cntt = cnt.reshape(E, 1)

    combT, maskT = pl.pallas_call(
        _dense_stage,
        grid=(NB,),
        in_specs=[
            pl.BlockSpec((1, BS), lambda b: (0, b)),
            pl.BlockSpec((1, BS), lambda b: (0, b)),
            pl.BlockSpec((1, BS), lambda b: (0, b)),
            pl.BlockSpec((1, BS), lambda b: (0, b)),
            pl.BlockSpec((1, BS), lambda b: (0, b)),
            pl.BlockSpec((1, BS), lambda b: (0, b)),
            pl.BlockSpec((E, 1), lambda b: (0, 0)),
        ],
        out_specs=[
            pl.BlockSpec((E, CAP, BS), lambda b: (0, 0, b)),
            pl.BlockSpec((E, CAP, BS), lambda b: (0, 0, b)),
        ],
        out_shape=[
            jax.ShapeDtypeStruct((E, CAP, S), jnp.float32),
            jax.ShapeDtypeStruct((E, CAP, S), jnp.int8),
        ],
        compiler_params=pltpu.CompilerParams(
            dimension_semantics=("arbitrary",)),
    )(e1t, e2t, g1t, g2t, loc1t, loc2pt, cntt)

    exp_counts = cnt.reshape(E)
    me = gsum.reshape(E) / S
    ce = exp_counts.astype(jnp.float32) / S
    l_aux = jnp.mean(me * ce) * E * E
    comb = jnp.transpose(combT, (2, 0, 1))
    mask = jnp.transpose(maskT, (2, 0, 1)).astype(bool)
    return (l_aux, comb, mask, exp_counts)


# speedup vs baseline: 3.3007x; 1.3183x over previous
import jax
import jax.numpy as jnp
from jax.experimental import pallas as pl
from jax.experimental.pallas import tpu as pltpu

S = 2048
HIDDEN = 4096
E = 64
CAP = 64
BS = 256
NB = S // BS
EPS = float(jnp.finfo(jnp.float32).eps)


def _gate_body(x_ref, w_ref, tri_ref,
               comb_ref, mask_ref, cnt_ref, gsum_ref,
               scr, carry1, carry2, gacc):
    p = pl.program_id(0)
    b = pl.program_id(1)

    @pl.when((p == 0) & (b == 0))
    def _init():
        carry1[...] = jnp.zeros_like(carry1)
        carry2[...] = jnp.zeros_like(carry2)
        gacc[...] = jnp.zeros_like(gacc)

    eio = jax.lax.broadcasted_iota(jnp.int32, (E, BS), 0)

    @pl.when(p == 0)
    def _gating():
        x = x_ref[...]
        w = w_ref[...]
        logits = jax.lax.dot_general(w, x, (((1,), (1,)), ((), ())),
                                     preferred_element_type=jnp.float32)
        m = jnp.max(logits, axis=0, keepdims=True)
        ex = jnp.exp(logits - m)
        gates = ex / jnp.sum(ex, axis=0, keepdims=True)

        g1 = jnp.max(gates, axis=0, keepdims=True)
        e1 = jnp.min(jnp.where(gates == g1, eio, E), axis=0, keepdims=True)
        hit1 = eio == e1
        gates_m = jnp.where(hit1, -1.0, gates)
        g2 = jnp.max(gates_m, axis=0, keepdims=True)
        e2 = jnp.min(jnp.where(gates_m == g2, eio, E), axis=0, keepdims=True)
        hit2 = eio == e2

        mask1 = hit1.astype(jnp.float32)
        mask2 = hit2.astype(jnp.float32)
        tri = tri_ref[...]
        cs1 = jax.lax.dot_general(mask1, tri, (((1,), (0,)), ((), ())),
                                  preferred_element_type=jnp.float32)
        cs2 = jax.lax.dot_general(mask2, tri, (((1,), (0,)), ((), ())),
                                  preferred_element_type=jnp.float32)
        c1 = carry1[...]
        c2 = carry2[...]
        loc1 = cs1 - 1.0 + c1
        loc2p = cs2 - 1.0 + c2

        ds = pl.ds(b * BS, BS)
        scr[0:1, ds] = e1.astype(jnp.float32)
        scr[1:2, ds] = e2.astype(jnp.float32)
        scr[2:3, ds] = g1
        scr[3:4, ds] = g2
        scr[4:5, ds] = jnp.sum(loc1 * mask1, axis=0, keepdims=True)
        scr[5:6, ds] = jnp.sum(loc2p * mask2, axis=0, keepdims=True)

        carry1[...] = c1 + cs1[:, BS - 1:BS]
        carry2[...] = c2 + cs2[:, BS - 1:BS]
        gacc[...] = gacc[...] + jnp.sum(gates, axis=1, keepdims=True)

    @pl.when(p == 1)
    def _materialize():
        ds = pl.ds(b * BS, BS)
        e1 = scr[0:1, ds].astype(jnp.int32)
        e2 = scr[1:2, ds].astype(jnp.int32)
        g1 = scr[2:3, ds]
        g2 = scr[3:4, ds]
        loc1f = scr[4:5, ds]
        loc2pf = scr[5:6, ds]
        cnt = carry1[...]

        cnt2 = jnp.sum(jnp.where(eio == e2, cnt, 0.0), axis=0, keepdims=True)
        loc2f = loc2pf + cnt2
        loc1 = loc1f.astype(jnp.int32)
        loc2 = loc2f.astype(jnp.int32)

        kept1 = loc1 < CAP
        kept2 = loc2 < CAP
        g1k = jnp.where(kept1, g1, 0.0)
        g2k = jnp.where(kept2, g2, 0.0)
        denom = jnp.maximum(g1k + g2k, EPS)
        c1 = g1k / denom
        c2 = g2k / denom

        cio = jax.lax.broadcasted_iota(jnp.int32, (CAP, BS), 0)
        a1 = jnp.where(eio == e1, c1, 0.0)
        a2 = jnp.where(eio == e2, c2, 0.0)
        b1 = (cio == loc1).astype(jnp.float32)
        b2 = (cio == loc2).astype(jnp.float32)
        comb = (a1[:, None, :] * b1[None, :, :]
                + a2[:, None, :] * b2[None, :, :])
        comb_ref[...] = comb
        mask_ref[...] = (comb != 0.0).astype(jnp.int8)
        cnt_ref[...] = cnt.astype(jnp.int32)
        gsum_ref[...] = gacc[...]


@jax.jit
def kernel(input, W):
    x = input.astype(jnp.float32)
    ti = jax.lax.broadcasted_iota(jnp.int32, (BS, BS), 0)
    si = jax.lax.broadcasted_iota(jnp.int32, (BS, BS), 1)
    tri = (ti <= si).astype(jnp.float32)

    combT, maskT, cnt, gsum = pl.pallas_call(
        _gate_body,
        grid=(2, NB),
        in_specs=[
            pl.BlockSpec((BS, HIDDEN), lambda p, b: (b * (1 - p), 0)),
            pl.BlockSpec((E, HIDDEN), lambda p, b: (0, 0)),
            pl.BlockSpec((BS, BS), lambda p, b: (0, 0)),
        ],
        out_specs=[
            pl.BlockSpec((E, CAP, BS), lambda p, b: (0, 0, b * p)),
            pl.BlockSpec((E, CAP, BS), lambda p, b: (0, 0, b * p)),
            pl.BlockSpec((E, 1), lambda p, b: (0, 0)),
            pl.BlockSpec((E, 1), lambda p, b: (0, 0)),
        ],
        out_shape=[
            jax.ShapeDtypeStruct((E, CAP, S), jnp.float32),
            jax.ShapeDtypeStruct((E, CAP, S), jnp.int8),
            jax.ShapeDtypeStruct((E, 1), jnp.int32),
            jax.ShapeDtypeStruct((E, 1), jnp.float32),
        ],
        scratch_shapes=[
            pltpu.VMEM((8, S), jnp.float32),
            pltpu.VMEM((E, 1), jnp.float32),
            pltpu.VMEM((E, 1), jnp.float32),
            pltpu.VMEM((E, 1), jnp.float32),
        ],
        compiler_params=pltpu.CompilerParams(
            dimension_semantics=("arbitrary", "arbitrary")),
    )(x, W, tri)

    exp_counts = cnt.reshape(E)
    me = gsum.reshape(E) / S
    ce = exp_counts.astype(jnp.float32) / S
    l_aux = jnp.mean(me * ce) * E * E
    comb = jnp.transpose(combT, (2, 0, 1))
    mask = jnp.transpose(maskT, (2, 0, 1)).astype(bool)
    return (l_aux, comb, mask, exp_counts)


# speedup vs baseline: 3.3459x; 1.0137x over previous
import jax
import jax.numpy as jnp
from jax.experimental import pallas as pl
from jax.experimental.pallas import tpu as pltpu

S = 2048
HIDDEN = 4096
E = 64
CAP = 64
BS = 256
NB = S // BS
EPS = float(jnp.finfo(jnp.float32).eps)


def _gate_body(x_ref, w_ref,
               comb_ref, mask_ref, cnt_ref, gsum_ref,
               scr, carry1, carry2, gacc):
    p = pl.program_id(0)
    b = pl.program_id(1)

    @pl.when((p == 0) & (b == 0))
    def _init():
        carry1[...] = jnp.zeros_like(carry1)
        carry2[...] = jnp.zeros_like(carry2)
        gacc[...] = jnp.zeros_like(gacc)

    eio = jax.lax.broadcasted_iota(jnp.int32, (E, BS), 0)

    @pl.when(p == 0)
    def _gating():
        x = x_ref[...]
        w = w_ref[...]
        logits = jax.lax.dot_general(w, x, (((1,), (1,)), ((), ())),
                                     preferred_element_type=jnp.float32)
        m = jnp.max(logits, axis=0, keepdims=True)
        ex = jnp.exp(logits - m)
        gates = ex / jnp.sum(ex, axis=0, keepdims=True)

        g1 = jnp.max(gates, axis=0, keepdims=True)
        e1 = jnp.min(jnp.where(gates == g1, eio, E), axis=0, keepdims=True)
        hit1 = eio == e1
        gates_m = jnp.where(hit1, -1.0, gates)
        g2 = jnp.max(gates_m, axis=0, keepdims=True)
        e2 = jnp.min(jnp.where(gates_m == g2, eio, E), axis=0, keepdims=True)
        hit2 = eio == e2

        mask1 = hit1.astype(jnp.float32)
        mask2 = hit2.astype(jnp.float32)
        ti = jax.lax.broadcasted_iota(jnp.int32, (BS, BS), 0)
        si = jax.lax.broadcasted_iota(jnp.int32, (BS, BS), 1)
        tri = (ti <= si).astype(jnp.float32)
        cs1 = jax.lax.dot_general(mask1, tri, (((1,), (0,)), ((), ())),
                                  preferred_element_type=jnp.float32)
        cs2 = jax.lax.dot_general(mask2, tri, (((1,), (0,)), ((), ())),
                                  preferred_element_type=jnp.float32)
        c1 = carry1[...]
        c2 = carry2[...]
        loc1 = cs1 - 1.0 + c1
        loc2p = cs2 - 1.0 + c2

        ds = pl.ds(b * BS, BS)
        scr[0:1, ds] = e1.astype(jnp.float32)
        scr[1:2, ds] = e2.astype(jnp.float32)
        scr[2:3, ds] = g1
        scr[3:4, ds] = g2
        scr[4:5, ds] = jnp.sum(loc1 * mask1, axis=0, keepdims=True)
        scr[5:6, ds] = jnp.sum(loc2p * mask2, axis=0, keepdims=True)

        carry1[...] = c1 + cs1[:, BS - 1:BS]
        carry2[...] = c2 + cs2[:, BS - 1:BS]
        gacc[...] = gacc[...] + jnp.sum(gates, axis=1, keepdims=True)

    @pl.when(p == 1)
    def _materialize():
        ds = pl.ds(b * BS, BS)
        e1 = scr[0:1, ds].astype(jnp.int32)
        e2 = scr[1:2, ds].astype(jnp.int32)
        g1 = scr[2:3, ds]
        g2 = scr[3:4, ds]
        loc1f = scr[4:5, ds]
        loc2pf = scr[5:6, ds]
        cnt = carry1[...]

        cnt2 = jnp.sum(jnp.where(eio == e2, cnt, 0.0), axis=0, keepdims=True)
        loc2f = loc2pf + cnt2
        loc1 = loc1f.astype(jnp.int32)
        loc2 = loc2f.astype(jnp.int32)

        kept1 = loc1 < CAP
        kept2 = loc2 < CAP
        g1k = jnp.where(kept1, g1, 0.0)
        g2k = jnp.where(kept2, g2, 0.0)
        denom = jnp.maximum(g1k + g2k, EPS)
        c1 = g1k / denom
        c2 = g2k / denom

        cio = jax.lax.broadcasted_iota(jnp.int32, (CAP, BS), 0)
        a1 = jnp.where(eio == e1, c1, 0.0)
        a2 = jnp.where(eio == e2, c2, 0.0)
        b1 = (cio == loc1).astype(jnp.float32)
        b2 = (cio == loc2).astype(jnp.float32)
        comb = (a1[:, None, :] * b1[None, :, :]
                + a2[:, None, :] * b2[None, :, :])
        comb_ref[...] = comb
        mask_ref[...] = (comb != 0.0).astype(jnp.int8)
        cnt_ref[...] = cnt.astype(jnp.int32)
        gsum_ref[...] = gacc[...]


@jax.jit
def kernel(input, W):
    x = input.astype(jnp.float32)

    combT, maskT, cnt, gsum = pl.pallas_call(
        _gate_body,
        grid=(2, NB),
        in_specs=[
            pl.BlockSpec((BS, HIDDEN), lambda p, b: (b * (1 - p), 0)),
            pl.BlockSpec((E, HIDDEN), lambda p, b: (0, 0)),
        ],
        out_specs=[
            pl.BlockSpec((E, CAP, BS), lambda p, b: (0, 0, b * p)),
            pl.BlockSpec((E, CAP, BS), lambda p, b: (0, 0, b * p)),
            pl.BlockSpec((E, 1), lambda p, b: (0, 0)),
            pl.BlockSpec((E, 1), lambda p, b: (0, 0)),
        ],
        out_shape=[
            jax.ShapeDtypeStruct((E, CAP, S), jnp.float32),
            jax.ShapeDtypeStruct((E, CAP, S), jnp.int8),
            jax.ShapeDtypeStruct((E, 1), jnp.int32),
            jax.ShapeDtypeStruct((E, 1), jnp.float32),
        ],
        scratch_shapes=[
            pltpu.VMEM((8, S), jnp.float32),
            pltpu.VMEM((E, 1), jnp.float32),
            pltpu.VMEM((E, 1), jnp.float32),
            pltpu.VMEM((E, 1), jnp.float32),
        ],
        compiler_params=pltpu.CompilerParams(
            dimension_semantics=("arbitrary", "arbitrary")),
    )(x, W)

    exp_counts = cnt.reshape(E)
    me = gsum.reshape(E) / S
    ce = exp_counts.astype(jnp.float32) / S
    l_aux = jnp.mean(me * ce) * E * E
    comb = jnp.transpose(combT, (2, 0, 1))
    mask = jnp.transpose(maskT, (2, 0, 1)).astype(bool)
    return (l_aux, comb, mask, exp_counts)


# speedup vs baseline: 3.4431x; 1.0291x over previous
import jax
import jax.numpy as jnp
from jax.experimental import pallas as pl
from jax.experimental.pallas import tpu as pltpu

S = 2048
HIDDEN = 4096
E = 64
CAP = 64
BS = 512
NB = S // BS
EPS = float(jnp.finfo(jnp.float32).eps)


def _gate_body(x_ref, w_ref,
               comb_ref, mask_ref, cnt_ref, gsum_ref,
               scr, carry1, carry2, gacc):
    p = pl.program_id(0)
    b = pl.program_id(1)

    @pl.when((p == 0) & (b == 0))
    def _init():
        carry1[...] = jnp.zeros_like(carry1)
        carry2[...] = jnp.zeros_like(carry2)
        gacc[...] = jnp.zeros_like(gacc)

    eio = jax.lax.broadcasted_iota(jnp.int32, (E, BS), 0)

    @pl.when(p == 0)
    def _gating():
        x = x_ref[...]
        w = w_ref[...]
        logits = jax.lax.dot_general(w, x, (((1,), (1,)), ((), ())),
                                     preferred_element_type=jnp.float32)
        m = jnp.max(logits, axis=0, keepdims=True)
        ex = jnp.exp(logits - m)
        gates = ex / jnp.sum(ex, axis=0, keepdims=True)

        g1 = jnp.max(gates, axis=0, keepdims=True)
        e1 = jnp.min(jnp.where(gates == g1, eio, E), axis=0, keepdims=True)
        hit1 = eio == e1
        gates_m = jnp.where(hit1, -1.0, gates)
        g2 = jnp.max(gates_m, axis=0, keepdims=True)
        e2 = jnp.min(jnp.where(gates_m == g2, eio, E), axis=0, keepdims=True)
        hit2 = eio == e2

        mask1 = hit1.astype(jnp.float32)
        mask2 = hit2.astype(jnp.float32)
        ti = jax.lax.broadcasted_iota(jnp.int32, (BS, BS), 0)
        si = jax.lax.broadcasted_iota(jnp.int32, (BS, BS), 1)
        tri = (ti <= si).astype(jnp.float32)
        cs1 = jax.lax.dot_general(mask1, tri, (((1,), (0,)), ((), ())),
                                  preferred_element_type=jnp.float32)
        cs2 = jax.lax.dot_general(mask2, tri, (((1,), (0,)), ((), ())),
                                  preferred_element_type=jnp.float32)
        c1 = carry1[...]
        c2 = carry2[...]
        loc1 = cs1 - 1.0 + c1
        loc2p = cs2 - 1.0 + c2

        ds = pl.ds(b * BS, BS)
        scr[0:1, ds] = e1.astype(jnp.float32)
        scr[1:2, ds] = e2.astype(jnp.float32)
        scr[2:3, ds] = g1
        scr[3:4, ds] = g2
        scr[4:5, ds] = jnp.sum(loc1 * mask1, axis=0, keepdims=True)
        scr[5:6, ds] = jnp.sum(loc2p * mask2, axis=0, keepdims=True)

        carry1[...] = c1 + cs1[:, BS - 1:BS]
        carry2[...] = c2 + cs2[:, BS - 1:BS]
        gacc[...] = gacc[...] + jnp.sum(gates, axis=1, keepdims=True)

    @pl.when(p == 1)
    def _materialize():
        ds = pl.ds(b * BS, BS)
        e1 = scr[0:1, ds].astype(jnp.int32)
        e2 = scr[1:2, ds].astype(jnp.int32)
        g1 = scr[2:3, ds]
        g2 = scr[3:4, ds]
        loc1f = scr[4:5, ds]
        loc2pf = scr[5:6, ds]
        cnt = carry1[...]

        cnt2 = jnp.sum(jnp.where(eio == e2, cnt, 0.0), axis=0, keepdims=True)
        loc2f = loc2pf + cnt2
        loc1 = loc1f.astype(jnp.int32)
        loc2 = loc2f.astype(jnp.int32)

        kept1 = loc1 < CAP
        kept2 = loc2 < CAP
        g1k = jnp.where(kept1, g1, 0.0)
        g2k = jnp.where(kept2, g2, 0.0)
        denom = jnp.maximum(g1k + g2k, EPS)
        c1 = g1k / denom
        c2 = g2k / denom

        cio = jax.lax.broadcasted_iota(jnp.int32, (CAP, BS), 0)
        a1 = jnp.where(eio == e1, c1, 0.0)
        a2 = jnp.where(eio == e2, c2, 0.0)
        b1 = (cio == loc1).astype(jnp.float32)
        b2 = (cio == loc2).astype(jnp.float32)
        comb = (a1[:, None, :] * b1[None, :, :]
                + a2[:, None, :] * b2[None, :, :])
        comb_ref[...] = comb
        mask_ref[...] = (comb != 0.0).astype(jnp.int8)
        cnt_ref[...] = cnt.astype(jnp.int32)
        gsum_ref[...] = gacc[...]


@jax.jit
def kernel(input, W):
    x = input.astype(jnp.float32)

    combT, maskT, cnt, gsum = pl.pallas_call(
        _gate_body,
        grid=(2, NB),
        in_specs=[
            pl.BlockSpec((BS, HIDDEN), lambda p, b: (b * (1 - p), 0)),
            pl.BlockSpec((E, HIDDEN), lambda p, b: (0, 0)),
        ],
        out_specs=[
            pl.BlockSpec((E, CAP, BS), lambda p, b: (0, 0, b * p)),
            pl.BlockSpec((E, CAP, BS), lambda p, b: (0, 0, b * p)),
            pl.BlockSpec((E, 1), lambda p, b: (0, 0)),
            pl.BlockSpec((E, 1), lambda p, b: (0, 0)),
        ],
        out_shape=[
            jax.ShapeDtypeStruct((E, CAP, S), jnp.float32),
            jax.ShapeDtypeStruct((E, CAP, S), jnp.int8),
            jax.ShapeDtypeStruct((E, 1), jnp.int32),
            jax.ShapeDtypeStruct((E, 1), jnp.float32),
        ],
        scratch_shapes=[
            pltpu.VMEM((8, S), jnp.float32),
            pltpu.VMEM((E, 1), jnp.float32),
            pltpu.VMEM((E, 1), jnp.float32),
            pltpu.VMEM((E, 1), jnp.float32),
        ],
        compiler_params=pltpu.CompilerParams(
            dimension_semantics=("arbitrary", "arbitrary")),
    )(x, W)

    exp_counts = cnt.reshape(E)
    me = gsum.reshape(E) / S
    ce = exp_counts.astype(jnp.float32) / S
    l_aux = jnp.mean(me * ce) * E * E
    comb = jnp.transpose(combT, (2, 0, 1))
    mask = jnp.transpose(maskT, (2, 0, 1)).astype(bool)
    return (l_aux, comb, mask, exp_counts)
